# Initial kernel scaffold; baseline (speedup 1.0000x reference)
import jax
import jax.numpy as jnp
from jax.experimental import pallas as pl


def kernel(input_ids, segment_ids, word_emb, seg_emb, pos_emb, ln_gamma, ln_beta):
    raise NotImplementedError("write your pallas kernel here")


# speedup vs baseline: 6.6026x; 6.6026x over previous
import functools

import jax
import jax.numpy as jnp
from jax import lax
from jax.experimental import pallas as pl
from jax.experimental.pallas import tpu as pltpu
from jax.experimental.pallas import tpu_sc as plsc

B, L, E = 1024, 200, 128
N = B * L
MAX_POS = 512
PAD_ID = 0
EPS = 1e-12
GATHER_W = 128
BBA = 128
BB = 16
TOK = BB * L


def _sc_gather_rows(table, idx_flat):
    mesh = plsc.VectorSubcoreMesh(core_axis_name="c", subcore_axis_name="s")

    @functools.partial(
        pl.kernel,
        out_type=jax.ShapeDtypeStruct((N, E), jnp.float32),
        mesh=mesh,
    )
    def gather_kernel(x_hbm, i_hbm, o_hbm):
        def body(i_vmem, o_vmem):
            pltpu.sync_copy(x_hbm.at[i_vmem.at[0]], o_vmem)

        pltpu.emit_pipeline(
            body,
            grid=(N // GATHER_W,),
            in_specs=[pl.BlockSpec((1, GATHER_W), lambda i: (0, i))],
            out_specs=[pl.BlockSpec((GATHER_W, E), lambda i: (i, 0))],
            core_axis_name=("c", "s"),
            dimension_semantics=(pltpu.PARALLEL,),
        )(i_hbm, o_hbm)

    return gather_kernel(table, idx_flat)


def _posid_kernel(ids_ref, tri_ref, pos_ref):
    ids = ids_ref[...]
    mask = ids != PAD_ID
    mbf = mask.astype(jnp.bfloat16)
    posf = lax.dot_general(mbf, tri_ref[...], (((1,), (0,)), ((), ())),
                           preferred_element_type=jnp.float32)
    pos_ref[...] = jnp.where(mask, posf.astype(jnp.int32), 0)


def _finish_kernel(w_ref, seg_ref, pos_ref, segemb_ref, hi_ref, lo_ref,
                   gamma_ref, beta_ref, out_ref):
    pos = pos_ref[0]
    onehot = (pos == lax.broadcasted_iota(jnp.int32, (1, MAX_POS), 1))
    onehot = onehot.astype(jnp.bfloat16)
    dims = (((1,), (0,)), ((), ()))
    d3 = lax.dot_general(onehot, hi_ref[...], dims,
                         preferred_element_type=jnp.float32)
    d3 = d3 + lax.dot_general(onehot, lo_ref[...], dims,
                              preferred_element_type=jnp.float32)

    segf = seg_ref[0].astype(jnp.float32)
    s0 = segemb_ref[0:1, :]
    d2 = s0 + segf * (segemb_ref[1:2, :] - s0)

    t = w_ref[0] + d2 + d3
    mean = jnp.mean(t, axis=1, keepdims=True)
    tcen = t - mean
    var = jnp.mean(tcen * tcen, axis=1, keepdims=True)
    y = tcen * lax.rsqrt(var + EPS) * gamma_ref[...] + beta_ref[...]
    out_ref[0] = y


def kernel(input_ids, segment_ids, word_emb, seg_emb, pos_emb, ln_gamma,
           ln_beta):
    ids32 = input_ids.astype(jnp.int32)
    segs32 = segment_ids.astype(jnp.int32)

    wrows = _sc_gather_rows(word_emb, ids32.reshape(1, N))

    tri = (lax.broadcasted_iota(jnp.int32, (L, L), 0)
           <= lax.broadcasted_iota(jnp.int32, (L, L), 1)).astype(jnp.bfloat16)
    pos_ids = pl.pallas_call(
        _posid_kernel,
        grid=(B // BBA,),
        in_specs=[
            pl.BlockSpec((BBA, L), lambda i: (i, 0)),
            pl.BlockSpec((L, L), lambda i: (0, 0)),
        ],
        out_specs=pl.BlockSpec((BBA, L), lambda i: (i, 0)),
        out_shape=jax.ShapeDtypeStruct((B, L), jnp.int32),
    )(ids32, tri)

    hi = pos_emb.astype(jnp.bfloat16)
    lo = (pos_emb - hi.astype(jnp.float32)).astype(jnp.bfloat16)

    nblk = B // BB
    out = pl.pallas_call(
        _finish_kernel,
        grid=(nblk,),
        in_specs=[
            pl.BlockSpec((1, TOK, E), lambda i: (i, 0, 0)),
            pl.BlockSpec((1, TOK, 1), lambda i: (i, 0, 0)),
            pl.BlockSpec((1, TOK, 1), lambda i: (i, 0, 0)),
            pl.BlockSpec((2, E), lambda i: (0, 0)),
            pl.BlockSpec((MAX_POS, E), lambda i: (0, 0)),
            pl.BlockSpec((MAX_POS, E), lambda i: (0, 0)),
            pl.BlockSpec((1, E), lambda i: (0, 0)),
            pl.BlockSpec((1, E), lambda i: (0, 0)),
        ],
        out_specs=pl.BlockSpec((1, TOK, E), lambda i: (i, 0, 0)),
        out_shape=jax.ShapeDtypeStruct((nblk, TOK, E), jnp.float32),
    )(wrows.reshape(nblk, TOK, E), segs32.reshape(nblk, TOK, 1),
      pos_ids.reshape(nblk, TOK, 1), seg_emb, hi, lo,
      ln_gamma.reshape(1, E), ln_beta.reshape(1, E))
    return out.reshape(B, L, E)
